# baseline (speedup 1.0000x reference)
import jax
import jax.numpy as jnp
from jax import lax
from jax.experimental import pallas as pl
from jax.experimental.pallas import tpu as pltpu
from jax.experimental.pallas import tpu_sc as plsc

_N = 100000
_E = 3200000
_D = 128
_H = 64
_C = 16
_CH2 = _C // 2
_K = 10
_ALPHA = 0.1

_NC = 2
_NS = 16

_CH = 512
_SUB = 4
_SUP = 100
_PER_TILE = _CH * _SUB * _SUP
_EPAD = _PER_TILE * _NS

_NPAD = 100352
_RPT = _NPAD // _NS
_VROWS = _NPAD * _C // 128
_VROWS_H = _VROWS // 2
_HWORDS = _NPAD * _CH2

_mesh = plsc.VectorSubcoreMesh(
    core_axis_name="c", subcore_axis_name="s", num_cores=_NC, num_subcores=_NS
)
_mesh1 = plsc.VectorSubcoreMesh(
    core_axis_name="c", subcore_axis_name="s", num_cores=1, num_subcores=_NS
)


def _edge_body(u_hbm, row_hbm, col_hbm, zero_hbm, s_out,
               ridx, cidx, rows, u_sp, acc, isem, gs0, gs1):
    s = lax.axis_index("s")
    gs = (gs0, gs1)
    base = s * _RPT
    pltpu.sync_copy(u_hbm.at[pl.ds(base, _RPT)], u_sp.at[pl.ds(base, _RPT)])
    pltpu.sync_copy(zero_hbm.at[pl.ds(base, _RPT)], acc.at[pl.ds(base, _RPT)])
    plsc.subcore_barrier()

    def g_start(pslot, j, b):
        pltpu.async_copy(u_sp.at[ridx.at[pslot, j]], rows.at[b], gs[b])

    def g_wait(pslot, j, b):
        pltpu.make_async_copy(
            u_sp.at[ridx.at[pslot, j]], rows.at[b], gs[b]
        ).wait()

    def s_sync(pslot, j, b):
        pltpu.sync_copy(rows.at[b], acc.at[cidx.at[pslot, j]], add=True)

    def idx_fetch(sup, pslot):
        pltpu.async_copy(row_hbm.at[s, sup], ridx.at[pslot], isem)
        pltpu.async_copy(col_hbm.at[s, sup], cidx.at[pslot], isem)

    def idx_wait(sup, pslot):
        pltpu.make_async_copy(row_hbm.at[s, sup], ridx.at[pslot],
                              isem).wait()
        pltpu.make_async_copy(col_hbm.at[s, sup], cidx.at[pslot],
                              isem).wait()

    def steps(sup, p):
        for j in range(_SUB):
            b = j % 2
            g_wait(p, j, b)
            s_sync(p, j, b)
            if j < _SUB - 2:
                g_start(p, j + 2, b)
            else:
                g_start(1 - p, j - (_SUB - 2), b)
            if j == 0:
                idx_fetch(sup + 1, 1 - p)
            if j == 1:
                idx_wait(sup + 1, 1 - p)

    idx_fetch(0, 0)
    idx_wait(0, 0)
    g_start(0, 0, 0)
    g_start(0, 1, 1)
    steps(0, 0)

    def super_body(sup, carry):
        p = sup % 2
        steps(sup, p)
        return carry

    lax.fori_loop(1, _SUP, super_body, 0)

    p_last = (_SUP - 1) % 2
    g_wait(1 - p_last, 0, 0)
    g_wait(1 - p_last, 1, 1)

    plsc.subcore_barrier()
    pltpu.sync_copy(acc.at[pl.ds(base, _RPT)], s_out.at[pl.ds(base, _RPT)])


_edge_kernel = pl.kernel(
    _edge_body,
    out_type=jax.ShapeDtypeStruct((_NPAD, _CH2), jnp.float32),
    mesh=_mesh1,
    scratch_types=[
        pltpu.VMEM((2, _SUB, _CH), jnp.int32),
        pltpu.VMEM((2, _SUB, _CH), jnp.int32),
        pltpu.VMEM((2, _CH, _CH2), jnp.float32),
        pltpu.VMEM_SHARED((_NPAD, _CH2), jnp.float32),
        pltpu.VMEM_SHARED((_NPAD, _CH2), jnp.float32),
        pltpu.SemaphoreType.DMA,
        pltpu.SemaphoreType.DMA,
        pltpu.SemaphoreType.DMA,
    ],
    compiler_params=pltpu.CompilerParams(use_tc_tiling_on_sc=False),
)


def _deg_body(col_hbm, ones_hbm, zero_hbm, s_out, cidx, ones_v, acc):
    c = lax.axis_index("c")
    s = lax.axis_index("s")
    base = s * _RPT
    pltpu.sync_copy(zero_hbm.at[pl.ds(base, _RPT)], acc.at[pl.ds(base, _RPT)])
    pltpu.sync_copy(ones_hbm, ones_v)
    plsc.subcore_barrier()

    def super_body(i, carry):
        pltpu.sync_copy(col_hbm.at[s, i], cidx)

        def sub_body(j, carry2):
            pltpu.sync_copy(ones_v, acc.at[cidx.at[j]], add=True)
            return carry2

        lax.fori_loop(0, _SUB, sub_body, 0)
        return carry

    lax.fori_loop(0, _SUP, super_body, 0)
    plsc.subcore_barrier()
    pltpu.sync_copy(acc.at[pl.ds(base, _RPT)], s_out.at[c, pl.ds(base, _RPT)])


_deg_kernel = pl.kernel(
    _deg_body,
    out_type=jax.ShapeDtypeStruct((_NC, _NPAD, _CH2), jnp.float32),
    mesh=_mesh,
    scratch_types=[
        pltpu.VMEM((_SUB, _CH), jnp.int32),
        pltpu.VMEM((_CH, _CH2), jnp.float32),
        pltpu.VMEM_SHARED((_NPAD, _CH2), jnp.float32),
    ],
    compiler_params=pltpu.CompilerParams(use_tc_tiling_on_sc=False),
)


_MLP_R = 1024


def _mlp_body(x_ref, w1_ref, b1_ref, w2_ref, b2_ref, h_ref):
    a1 = jnp.dot(x_ref[...], w1_ref[...], preferred_element_type=jnp.float32)
    a1 = jnp.maximum(a1 + b1_ref[0:1, :], 0.0)
    h_ref[...] = (
        jnp.dot(a1, w2_ref[...], preferred_element_type=jnp.float32)
        + b2_ref[0:1, :]
    )


_mlp_kernel = pl.pallas_call(
    _mlp_body,
    grid=(_NPAD // _MLP_R,),
    in_specs=[
        pl.BlockSpec((_MLP_R, _D), lambda i: (i, 0)),
        pl.BlockSpec((_D, _H), lambda i: (0, 0)),
        pl.BlockSpec((8, _H), lambda i: (0, 0)),
        pl.BlockSpec((_H, _C), lambda i: (0, 0)),
        pl.BlockSpec((8, _C), lambda i: (0, 0)),
    ],
    out_specs=pl.BlockSpec((_MLP_R, _C), lambda i: (i, 0)),
    out_shape=jax.ShapeDtypeStruct((_NPAD, _C), jnp.float32),
)


_VB = 392


def _prep_body(sdeg_ref, h_ref, dinv_ref, a_ref, bh_ref, u0_ref):
    i = pl.program_id(0)
    deg = sdeg_ref[...] + 1.0
    dinv = lax.rsqrt(deg)
    fl = (lax.broadcasted_iota(jnp.int32, (_VB, 128), 0) + i * _VB) * 128 \
        + lax.broadcasted_iota(jnp.int32, (_VB, 128), 1)
    node = (fl % _HWORDS) // _CH2
    mask = node < _N
    dinv_ref[...] = dinv
    a_ref[...] = jnp.where(mask, (1.0 - _ALPHA) * dinv * dinv, 0.0)
    h = h_ref[...]
    bh_ref[...] = jnp.where(mask, _ALPHA * dinv * h, 0.0)
    u0_ref[...] = jnp.where(mask, dinv * h, 0.0)


_prep_kernel = pl.pallas_call(
    _prep_body,
    grid=(_VROWS // _VB,),
    in_specs=[pl.BlockSpec((_VB, 128), lambda i: (i, 0))] * 2,
    out_specs=[pl.BlockSpec((_VB, 128), lambda i: (i, 0))] * 4,
    out_shape=[jax.ShapeDtypeStruct((_VROWS, 128), jnp.float32)] * 4,
)


def _combine_body(s_ref, u_ref, a_ref, bh_ref, o_ref):
    o_ref[...] = a_ref[...] * (s_ref[...] + u_ref[...]) + bh_ref[...]


_combine_kernel = pl.pallas_call(
    _combine_body,
    grid=(_VROWS_H // _VB,),
    in_specs=[pl.BlockSpec((_VB, 128), lambda i: (i, 0))] * 4,
    out_specs=pl.BlockSpec((_VB, 128), lambda i: (i, 0)),
    out_shape=jax.ShapeDtypeStruct((_VROWS_H, 128), jnp.float32),
)


_FB = 1024


def _final_body(s0_ref, s1_ref, u0_ref, u1_ref, d0_ref, d1_ref,
                h0_ref, h1_ref, o_ref):
    t0 = (1.0 - _ALPHA) * d0_ref[...] * (s0_ref[...] + u0_ref[...]) \
        + _ALPHA * h0_ref[...]
    t1 = (1.0 - _ALPHA) * d1_ref[...] * (s1_ref[...] + u1_ref[...]) \
        + _ALPHA * h1_ref[...]
    t = jnp.concatenate([t0, t1], axis=1)
    m = jnp.max(t, axis=1, keepdims=True)
    e = jnp.exp(t - m)
    lse = jnp.log(jnp.sum(e, axis=1, keepdims=True))
    o_ref[...] = t - m - lse


_final_kernel = pl.pallas_call(
    _final_body,
    grid=(_NPAD // _FB,),
    in_specs=[pl.BlockSpec((_FB, _CH2), lambda i: (i, 0))] * 8,
    out_specs=pl.BlockSpec((_FB, _C), lambda i: (i, 0)),
    out_shape=jax.ShapeDtypeStruct((_NPAD, _C), jnp.float32),
)


def kernel(x, edge_index, W1, b1, W2, b2):
    row = edge_index[0].astype(jnp.int32)
    col = edge_index[1].astype(jnp.int32)
    pad = _EPAD - _E
    fill = jnp.full((pad,), _N, dtype=jnp.int32)
    tile_fill = jnp.full((_NS, _SUB * _CH), _N, dtype=jnp.int32)

    def _pack(idx):
        t = jnp.concatenate([idx, fill]).reshape(_NS, _SUP * _SUB * _CH)
        t = jnp.concatenate([t, tile_fill], axis=1)
        return t.reshape(_NS, _SUP + 1, _SUB, _CH)

    rowp = _pack(row)
    colp = _pack(col)

    xp = jnp.concatenate(
        [x, jnp.zeros((_NPAD - _N, _D), dtype=jnp.float32)], axis=0
    )
    b1b = jnp.broadcast_to(b1.reshape(1, _H), (8, _H))
    b2b = jnp.broadcast_to(b2.reshape(1, _C), (8, _C))
    h = _mlp_kernel(xp, W1, b1b, W2, b2b)
    hs = h.reshape(_NPAD, _NC, _CH2).transpose(1, 0, 2)

    zeros = jnp.zeros((_NPAD, _CH2), dtype=jnp.float32)
    ones_t = jnp.ones((_CH, _CH2), dtype=jnp.float32)

    sdeg = _deg_kernel(colp, ones_t, zeros)
    dinv_v, a_v, bh_v, u_v = _prep_kernel(
        sdeg.reshape(_VROWS, 128), hs.reshape(_VROWS, 128)
    )

    a_h = (a_v[:_VROWS_H], a_v[_VROWS_H:])
    bh_h = (bh_v[:_VROWS_H], bh_v[_VROWS_H:])
    u_h = [u_v[:_VROWS_H], u_v[_VROWS_H:]]
    for _ in range(_K - 1):
        sv = [_edge_kernel(u_h[c].reshape(_NPAD, _CH2), rowp, colp, zeros)
              for c in range(_NC)]
        for c in range(_NC):
            u_h[c] = _combine_kernel(
                sv[c].reshape(_VROWS_H, 128), u_h[c], a_h[c], bh_h[c]
            )

    sv = [_edge_kernel(u_h[c].reshape(_NPAD, _CH2), rowp, colp, zeros)
          for c in range(_NC)]
    ds = dinv_v.reshape(_NC, _NPAD, _CH2)
    out = _final_kernel(
        sv[0], sv[1],
        u_h[0].reshape(_NPAD, _CH2), u_h[1].reshape(_NPAD, _CH2),
        ds[0], ds[1], hs[0], hs[1]
    )
    return out[:_N]


# speedup vs baseline: 1.6179x; 1.6179x over previous
import jax
import jax.numpy as jnp
from jax import lax
from jax.experimental import pallas as pl
from jax.experimental.pallas import tpu as pltpu
from jax.experimental.pallas import tpu_sc as plsc

_N = 100000
_E = 3200000
_D = 128
_H = 64
_C = 16
_CH2 = _C // 2
_K = 10
_ALPHA = 0.1

_NC = 2
_NS = 16

_CH = 640
_SUB = 4
_SUP = 80
_PER_TILE = _CH * _SUB * _SUP
_EPAD = _PER_TILE * _NS

_NPAD = 100352
_RPT = _NPAD // _NS
_VROWS = _NPAD * _C // 128
_HWORDS = _NPAD * _CH2

_mesh = plsc.VectorSubcoreMesh(
    core_axis_name="c", subcore_axis_name="s", num_cores=_NC, num_subcores=_NS
)


def _edge_body(u_hbm, row_hbm, col_hbm, zero_hbm, s_out,
               ridx, cidx, rows, u_sp, acc, isem, gs0, gs1):
    c = lax.axis_index("c")
    s = lax.axis_index("s")
    gs = (gs0, gs1)
    base = s * _RPT
    pltpu.sync_copy(u_hbm.at[c, pl.ds(base, _RPT)], u_sp.at[pl.ds(base, _RPT)])
    pltpu.sync_copy(zero_hbm.at[pl.ds(base, _RPT)], acc.at[pl.ds(base, _RPT)])
    plsc.subcore_barrier()

    def g_start(pslot, j, b):
        pltpu.async_copy(u_sp.at[ridx.at[pslot, j]], rows.at[b], gs[b])

    def g_wait(pslot, j, b):
        pltpu.make_async_copy(
            u_sp.at[ridx.at[pslot, j]], rows.at[b], gs[b]
        ).wait()

    def s_sync(pslot, j, b):
        pltpu.sync_copy(rows.at[b], acc.at[cidx.at[pslot, j]], add=True)

    def idx_fetch(sup, pslot):
        pltpu.async_copy(row_hbm.at[s, sup], ridx.at[pslot], isem)
        pltpu.async_copy(col_hbm.at[s, sup], cidx.at[pslot], isem)

    def idx_wait(sup, pslot):
        pltpu.make_async_copy(row_hbm.at[s, sup], ridx.at[pslot],
                              isem).wait()
        pltpu.make_async_copy(col_hbm.at[s, sup], cidx.at[pslot],
                              isem).wait()

    def steps(sup, p):
        for j in range(_SUB):
            b = j % 2
            g_wait(p, j, b)
            s_sync(p, j, b)
            if j < _SUB - 2:
                g_start(p, j + 2, b)
            else:
                g_start(1 - p, j - (_SUB - 2), b)
            if j == 0:
                idx_fetch(sup + 1, 1 - p)
            if j == 1:
                idx_wait(sup + 1, 1 - p)

    idx_fetch(0, 0)
    idx_wait(0, 0)
    g_start(0, 0, 0)
    g_start(0, 1, 1)
    steps(0, 0)

    def super_body(sup, carry):
        p = sup % 2
        steps(sup, p)
        return carry

    lax.fori_loop(1, _SUP, super_body, 0)

    p_last = (_SUP - 1) % 2
    g_wait(1 - p_last, 0, 0)
    g_wait(1 - p_last, 1, 1)

    plsc.subcore_barrier()
    pltpu.sync_copy(acc.at[pl.ds(base, _RPT)], s_out.at[c, pl.ds(base, _RPT)])


_edge_kernel = pl.kernel(
    _edge_body,
    out_type=jax.ShapeDtypeStruct((_NC, _NPAD, _CH2), jnp.float32),
    mesh=_mesh,
    scratch_types=[
        pltpu.VMEM((2, _SUB, _CH), jnp.int32),
        pltpu.VMEM((2, _SUB, _CH), jnp.int32),
        pltpu.VMEM((2, _CH, _CH2), jnp.float32),
        pltpu.VMEM_SHARED((_NPAD, _CH2), jnp.float32),
        pltpu.VMEM_SHARED((_NPAD, _CH2), jnp.float32),
        pltpu.SemaphoreType.DMA,
        pltpu.SemaphoreType.DMA,
        pltpu.SemaphoreType.DMA,
    ],
    compiler_params=pltpu.CompilerParams(use_tc_tiling_on_sc=False),
)


def _deg_body(col_hbm, ones_hbm, zero_hbm, s_out, cidx, ones_v, acc):
    c = lax.axis_index("c")
    s = lax.axis_index("s")
    base = s * _RPT
    pltpu.sync_copy(zero_hbm.at[pl.ds(base, _RPT)], acc.at[pl.ds(base, _RPT)])
    pltpu.sync_copy(ones_hbm, ones_v)
    plsc.subcore_barrier()

    def super_body(i, carry):
        pltpu.sync_copy(col_hbm.at[s, i], cidx)

        def sub_body(j, carry2):
            pltpu.sync_copy(ones_v, acc.at[cidx.at[j]], add=True)
            return carry2

        lax.fori_loop(0, _SUB, sub_body, 0)
        return carry

    lax.fori_loop(0, _SUP, super_body, 0)
    plsc.subcore_barrier()
    pltpu.sync_copy(acc.at[pl.ds(base, _RPT)], s_out.at[c, pl.ds(base, _RPT)])


_deg_kernel = pl.kernel(
    _deg_body,
    out_type=jax.ShapeDtypeStruct((_NC, _NPAD, _CH2), jnp.float32),
    mesh=_mesh,
    scratch_types=[
        pltpu.VMEM((_SUB, _CH), jnp.int32),
        pltpu.VMEM((_CH, _CH2), jnp.float32),
        pltpu.VMEM_SHARED((_NPAD, _CH2), jnp.float32),
    ],
    compiler_params=pltpu.CompilerParams(use_tc_tiling_on_sc=False),
)


_MLP_R = 1024


def _mlp_body(x_ref, w1_ref, b1_ref, w2_ref, b2_ref, h_ref):
    a1 = jnp.dot(x_ref[...], w1_ref[...], preferred_element_type=jnp.float32)
    a1 = jnp.maximum(a1 + b1_ref[0:1, :], 0.0)
    h_ref[...] = (
        jnp.dot(a1, w2_ref[...], preferred_element_type=jnp.float32)
        + b2_ref[0:1, :]
    )


_mlp_kernel = pl.pallas_call(
    _mlp_body,
    grid=(_NPAD // _MLP_R,),
    in_specs=[
        pl.BlockSpec((_MLP_R, _D), lambda i: (i, 0)),
        pl.BlockSpec((_D, _H), lambda i: (0, 0)),
        pl.BlockSpec((8, _H), lambda i: (0, 0)),
        pl.BlockSpec((_H, _C), lambda i: (0, 0)),
        pl.BlockSpec((8, _C), lambda i: (0, 0)),
    ],
    out_specs=pl.BlockSpec((_MLP_R, _C), lambda i: (i, 0)),
    out_shape=jax.ShapeDtypeStruct((_NPAD, _C), jnp.float32),
)


_VB = 392


def _prep_body(sdeg_ref, h_ref, dinv_ref, a_ref, bh_ref, u0_ref):
    i = pl.program_id(0)
    deg = sdeg_ref[...] + 1.0
    dinv = lax.rsqrt(deg)
    fl = (lax.broadcasted_iota(jnp.int32, (_VB, 128), 0) + i * _VB) * 128 \
        + lax.broadcasted_iota(jnp.int32, (_VB, 128), 1)
    node = (fl % _HWORDS) // _CH2
    mask = node < _N
    dinv_ref[...] = dinv
    a_ref[...] = jnp.where(mask, (1.0 - _ALPHA) * dinv * dinv, 0.0)
    h = h_ref[...]
    bh_ref[...] = jnp.where(mask, _ALPHA * dinv * h, 0.0)
    u0_ref[...] = jnp.where(mask, dinv * h, 0.0)


_prep_kernel = pl.pallas_call(
    _prep_body,
    grid=(_VROWS // _VB,),
    in_specs=[pl.BlockSpec((_VB, 128), lambda i: (i, 0))] * 2,
    out_specs=[pl.BlockSpec((_VB, 128), lambda i: (i, 0))] * 4,
    out_shape=[jax.ShapeDtypeStruct((_VROWS, 128), jnp.float32)] * 4,
)


def _combine_body(s_ref, u_ref, a_ref, bh_ref, o_ref):
    o_ref[...] = a_ref[...] * (s_ref[...] + u_ref[...]) + bh_ref[...]


_combine_kernel = pl.pallas_call(
    _combine_body,
    grid=(_VROWS // _VB,),
    in_specs=[pl.BlockSpec((_VB, 128), lambda i: (i, 0))] * 4,
    out_specs=pl.BlockSpec((_VB, 128), lambda i: (i, 0)),
    out_shape=jax.ShapeDtypeStruct((_VROWS, 128), jnp.float32),
)


_FB = 1024


def _final_body(s0_ref, s1_ref, u0_ref, u1_ref, d0_ref, d1_ref,
                h0_ref, h1_ref, o_ref):
    t0 = (1.0 - _ALPHA) * d0_ref[...] * (s0_ref[...] + u0_ref[...]) \
        + _ALPHA * h0_ref[...]
    t1 = (1.0 - _ALPHA) * d1_ref[...] * (s1_ref[...] + u1_ref[...]) \
        + _ALPHA * h1_ref[...]
    t = jnp.concatenate([t0, t1], axis=1)
    m = jnp.max(t, axis=1, keepdims=True)
    e = jnp.exp(t - m)
    lse = jnp.log(jnp.sum(e, axis=1, keepdims=True))
    o_ref[...] = t - m - lse


_final_kernel = pl.pallas_call(
    _final_body,
    grid=(_NPAD // _FB,),
    in_specs=[pl.BlockSpec((_FB, _CH2), lambda i: (i, 0))] * 8,
    out_specs=pl.BlockSpec((_FB, _C), lambda i: (i, 0)),
    out_shape=jax.ShapeDtypeStruct((_NPAD, _C), jnp.float32),
)


def kernel(x, edge_index, W1, b1, W2, b2):
    row = edge_index[0].astype(jnp.int32)
    col = edge_index[1].astype(jnp.int32)
    pad = _EPAD - _E
    fill = jnp.full((pad,), _N, dtype=jnp.int32)
    tile_fill = jnp.full((_NS, _SUB * _CH), _N, dtype=jnp.int32)

    def _pack(idx):
        t = jnp.concatenate([idx, fill]).reshape(_NS, _SUP * _SUB * _CH)
        t = jnp.concatenate([t, tile_fill], axis=1)
        return t.reshape(_NS, _SUP + 1, _SUB, _CH)

    rowp = _pack(row)
    colp = _pack(col)

    xp = jnp.concatenate(
        [x, jnp.zeros((_NPAD - _N, _D), dtype=jnp.float32)], axis=0
    )
    b1b = jnp.broadcast_to(b1.reshape(1, _H), (8, _H))
    b2b = jnp.broadcast_to(b2.reshape(1, _C), (8, _C))
    h = _mlp_kernel(xp, W1, b1b, W2, b2b)
    hs = h.reshape(_NPAD, _NC, _CH2).transpose(1, 0, 2)

    zeros = jnp.zeros((_NPAD, _CH2), dtype=jnp.float32)
    ones_t = jnp.ones((_CH, _CH2), dtype=jnp.float32)

    sdeg = _deg_kernel(colp, ones_t, zeros)
    dinv_v, a_v, bh_v, u_v = _prep_kernel(
        sdeg.reshape(_VROWS, 128), hs.reshape(_VROWS, 128)
    )

    for _ in range(_K - 1):
        sv = _edge_kernel(u_v.reshape(_NC, _NPAD, _CH2), rowp, colp, zeros)
        u_v = _combine_kernel(sv.reshape(_VROWS, 128), u_v, a_v, bh_v)

    sv = _edge_kernel(u_v.reshape(_NC, _NPAD, _CH2), rowp, colp, zeros)
    us = u_v.reshape(_NC, _NPAD, _CH2)
    ds = dinv_v.reshape(_NC, _NPAD, _CH2)
    out = _final_kernel(
        sv[0], sv[1], us[0], us[1], ds[0], ds[1], hs[0], hs[1]
    )
    return out[:_N]


# speedup vs baseline: 1.6225x; 1.0028x over previous
import jax
import jax.numpy as jnp
from jax import lax
from jax.experimental import pallas as pl
from jax.experimental.pallas import tpu as pltpu
from jax.experimental.pallas import tpu_sc as plsc

_N = 100000
_E = 3200000
_D = 128
_H = 64
_C = 16
_CH2 = _C // 2
_K = 10
_ALPHA = 0.1

_NC = 2
_NS = 16

_CH = 640
_SUB = 4
_SUP = 80
_PER_TILE = _CH * _SUB * _SUP
_EPAD = _PER_TILE * _NS

_NPAD = 100352
_RPT = _NPAD // _NS
_VROWS = _NPAD * _C // 128
_HWORDS = _NPAD * _CH2

_mesh = plsc.VectorSubcoreMesh(
    core_axis_name="c", subcore_axis_name="s", num_cores=_NC, num_subcores=_NS
)


def _edge_body(u_hbm, row_hbm, col_hbm, zero_hbm, s_out,
               ridx, cidx, rows, u_sp, acc, isem, gs0, gs1):
    c = lax.axis_index("c")
    s = lax.axis_index("s")
    gs = (gs0, gs1)
    base = s * _RPT
    ubase = c * _NPAD + base
    pltpu.async_copy(u_hbm.at[pl.ds(ubase, _RPT)],
                     u_sp.at[pl.ds(base, _RPT)], gs0)
    pltpu.async_copy(zero_hbm.at[pl.ds(base, _RPT)],
                     acc.at[pl.ds(base, _RPT)], gs1)
    pltpu.async_copy(row_hbm.at[s, 0], ridx.at[0], isem)
    pltpu.async_copy(col_hbm.at[s, 0], cidx.at[0], isem)
    pltpu.make_async_copy(u_hbm.at[pl.ds(ubase, _RPT)],
                          u_sp.at[pl.ds(base, _RPT)], gs0).wait()
    pltpu.make_async_copy(zero_hbm.at[pl.ds(base, _RPT)],
                          acc.at[pl.ds(base, _RPT)], gs1).wait()
    plsc.subcore_barrier()

    def g_start(pslot, j, b):
        pltpu.async_copy(u_sp.at[ridx.at[pslot, j]], rows.at[b], gs[b])

    def g_wait(pslot, j, b):
        pltpu.make_async_copy(
            u_sp.at[ridx.at[pslot, j]], rows.at[b], gs[b]
        ).wait()

    def s_sync(pslot, j, b):
        pltpu.sync_copy(rows.at[b], acc.at[cidx.at[pslot, j]], add=True)

    def idx_fetch(sup, pslot):
        pltpu.async_copy(row_hbm.at[s, sup], ridx.at[pslot], isem)
        pltpu.async_copy(col_hbm.at[s, sup], cidx.at[pslot], isem)

    def idx_wait(sup, pslot):
        pltpu.make_async_copy(row_hbm.at[s, sup], ridx.at[pslot],
                              isem).wait()
        pltpu.make_async_copy(col_hbm.at[s, sup], cidx.at[pslot],
                              isem).wait()

    def steps(sup, p):
        for j in range(_SUB):
            b = j % 2
            g_wait(p, j, b)
            s_sync(p, j, b)
            if j < _SUB - 2:
                g_start(p, j + 2, b)
            else:
                g_start(1 - p, j - (_SUB - 2), b)
            if j == 0:
                idx_fetch(sup + 1, 1 - p)
            if j == 1:
                idx_wait(sup + 1, 1 - p)

    idx_wait(0, 0)
    g_start(0, 0, 0)
    g_start(0, 1, 1)
    steps(0, 0)

    def super_body(sup, carry):
        p = sup % 2
        steps(sup, p)
        return carry

    lax.fori_loop(1, _SUP, super_body, 0)

    p_last = (_SUP - 1) % 2
    g_wait(1 - p_last, 0, 0)
    g_wait(1 - p_last, 1, 1)

    plsc.subcore_barrier()
    pltpu.sync_copy(acc.at[pl.ds(base, _RPT)], s_out.at[c, pl.ds(base, _RPT)])


_edge_kernel = pl.kernel(
    _edge_body,
    out_type=jax.ShapeDtypeStruct((_NC, _NPAD, _CH2), jnp.float32),
    mesh=_mesh,
    scratch_types=[
        pltpu.VMEM((2, _SUB, _CH), jnp.int32),
        pltpu.VMEM((2, _SUB, _CH), jnp.int32),
        pltpu.VMEM((2, _CH, _CH2), jnp.float32),
        pltpu.VMEM_SHARED((_NPAD, _CH2), jnp.float32),
        pltpu.VMEM_SHARED((_NPAD, _CH2), jnp.float32),
        pltpu.SemaphoreType.DMA,
        pltpu.SemaphoreType.DMA,
        pltpu.SemaphoreType.DMA,
    ],
    compiler_params=pltpu.CompilerParams(use_tc_tiling_on_sc=False),
)


def _deg_body(col_hbm, ones_hbm, zero_hbm, s_out, cidx, ones_v, acc):
    c = lax.axis_index("c")
    s = lax.axis_index("s")
    base = s * _RPT
    pltpu.sync_copy(zero_hbm.at[pl.ds(base, _RPT)], acc.at[pl.ds(base, _RPT)])
    pltpu.sync_copy(ones_hbm, ones_v)
    plsc.subcore_barrier()

    def super_body(i, carry):
        pltpu.sync_copy(col_hbm.at[s, i], cidx)

        def sub_body(j, carry2):
            pltpu.sync_copy(ones_v, acc.at[cidx.at[j]], add=True)
            return carry2

        lax.fori_loop(0, _SUB, sub_body, 0)
        return carry

    lax.fori_loop(0, _SUP, super_body, 0)
    plsc.subcore_barrier()
    pltpu.sync_copy(acc.at[pl.ds(base, _RPT)], s_out.at[c, pl.ds(base, _RPT)])


_deg_kernel = pl.kernel(
    _deg_body,
    out_type=jax.ShapeDtypeStruct((_NC, _NPAD, _CH2), jnp.float32),
    mesh=_mesh,
    scratch_types=[
        pltpu.VMEM((_SUB, _CH), jnp.int32),
        pltpu.VMEM((_CH, _CH2), jnp.float32),
        pltpu.VMEM_SHARED((_NPAD, _CH2), jnp.float32),
    ],
    compiler_params=pltpu.CompilerParams(use_tc_tiling_on_sc=False),
)


_MLP_R = 1024


def _mlp_body(x_ref, w1_ref, b1_ref, w2_ref, b2_ref, h_ref):
    a1 = jnp.dot(x_ref[...], w1_ref[...], preferred_element_type=jnp.float32)
    a1 = jnp.maximum(a1 + b1_ref[0:1, :], 0.0)
    h_ref[...] = (
        jnp.dot(a1, w2_ref[...], preferred_element_type=jnp.float32)
        + b2_ref[0:1, :]
    )


_mlp_kernel = pl.pallas_call(
    _mlp_body,
    grid=(_NPAD // _MLP_R,),
    in_specs=[
        pl.BlockSpec((_MLP_R, _D), lambda i: (i, 0)),
        pl.BlockSpec((_D, _H), lambda i: (0, 0)),
        pl.BlockSpec((8, _H), lambda i: (0, 0)),
        pl.BlockSpec((_H, _C), lambda i: (0, 0)),
        pl.BlockSpec((8, _C), lambda i: (0, 0)),
    ],
    out_specs=pl.BlockSpec((_MLP_R, _C), lambda i: (i, 0)),
    out_shape=jax.ShapeDtypeStruct((_NPAD, _C), jnp.float32),
)


_VB = 392


def _prep_body(sdeg_ref, h_ref, dinv_ref, a_ref, bh_ref, u0_ref):
    i = pl.program_id(0)
    deg = sdeg_ref[...] + 1.0
    dinv = lax.rsqrt(deg)
    fl = (lax.broadcasted_iota(jnp.int32, (_VB, 128), 0) + i * _VB) * 128 \
        + lax.broadcasted_iota(jnp.int32, (_VB, 128), 1)
    node = (fl % _HWORDS) // _CH2
    mask = node < _N
    dinv_ref[...] = dinv
    a_ref[...] = jnp.where(mask, (1.0 - _ALPHA) * dinv * dinv, 0.0)
    h = h_ref[...]
    bh_ref[...] = jnp.where(mask, _ALPHA * dinv * h, 0.0)
    u0_ref[...] = jnp.where(mask, dinv * h, 0.0)


_prep_kernel = pl.pallas_call(
    _prep_body,
    grid=(_VROWS // _VB,),
    in_specs=[pl.BlockSpec((_VB, 128), lambda i: (i, 0))] * 2,
    out_specs=[pl.BlockSpec((_VB, 128), lambda i: (i, 0))] * 4,
    out_shape=[jax.ShapeDtypeStruct((_VROWS, 128), jnp.float32)] * 4,
)


def _combine_body(s_ref, u_ref, a_ref, bh_ref, o_ref):
    o_ref[...] = a_ref[...] * (s_ref[...] + u_ref[...]) + bh_ref[...]


_combine_kernel = pl.pallas_call(
    _combine_body,
    grid=(_VROWS // _VB,),
    in_specs=[pl.BlockSpec((_VB, 128), lambda i: (i, 0))] * 4,
    out_specs=pl.BlockSpec((_VB, 128), lambda i: (i, 0)),
    out_shape=jax.ShapeDtypeStruct((_VROWS, 128), jnp.float32),
)


_FB = 1024


def _final_body(s0_ref, s1_ref, u0_ref, u1_ref, d0_ref, d1_ref,
                h0_ref, h1_ref, o_ref):
    t0 = (1.0 - _ALPHA) * d0_ref[...] * (s0_ref[...] + u0_ref[...]) \
        + _ALPHA * h0_ref[...]
    t1 = (1.0 - _ALPHA) * d1_ref[...] * (s1_ref[...] + u1_ref[...]) \
        + _ALPHA * h1_ref[...]
    t = jnp.concatenate([t0, t1], axis=1)
    m = jnp.max(t, axis=1, keepdims=True)
    e = jnp.exp(t - m)
    lse = jnp.log(jnp.sum(e, axis=1, keepdims=True))
    o_ref[...] = t - m - lse


_final_kernel = pl.pallas_call(
    _final_body,
    grid=(_NPAD // _FB,),
    in_specs=[pl.BlockSpec((_FB, _CH2), lambda i: (i, 0))] * 8,
    out_specs=pl.BlockSpec((_FB, _C), lambda i: (i, 0)),
    out_shape=jax.ShapeDtypeStruct((_NPAD, _C), jnp.float32),
)


def kernel(x, edge_index, W1, b1, W2, b2):
    row = edge_index[0].astype(jnp.int32)
    col = edge_index[1].astype(jnp.int32)
    pad = _EPAD - _E
    fill = jnp.full((pad,), _N, dtype=jnp.int32)
    tile_fill = jnp.full((_NS, _SUB * _CH), _N, dtype=jnp.int32)

    def _pack(idx):
        t = jnp.concatenate([idx, fill]).reshape(_NS, _SUP * _SUB * _CH)
        t = jnp.concatenate([t, tile_fill], axis=1)
        return t.reshape(_NS, _SUP + 1, _SUB, _CH)

    rowp = _pack(row)
    colp = _pack(col)

    xp = jnp.concatenate(
        [x, jnp.zeros((_NPAD - _N, _D), dtype=jnp.float32)], axis=0
    )
    b1b = jnp.broadcast_to(b1.reshape(1, _H), (8, _H))
    b2b = jnp.broadcast_to(b2.reshape(1, _C), (8, _C))
    h = _mlp_kernel(xp, W1, b1b, W2, b2b)
    hs = h.reshape(_NPAD, _NC, _CH2).transpose(1, 0, 2)

    zeros = jnp.zeros((_NPAD, _CH2), dtype=jnp.float32)
    ones_t = jnp.ones((_CH, _CH2), dtype=jnp.float32)

    sdeg = _deg_kernel(colp, ones_t, zeros)
    dinv_v, a_v, bh_v, u_v = _prep_kernel(
        sdeg.reshape(_VROWS, 128), hs.reshape(_VROWS, 128)
    )

    for _ in range(_K - 1):
        sv = _edge_kernel(u_v.reshape(_NC * _NPAD, _CH2), rowp, colp, zeros)
        u_v = _combine_kernel(sv.reshape(_VROWS, 128), u_v, a_v, bh_v)

    sv = _edge_kernel(u_v.reshape(_NC * _NPAD, _CH2), rowp, colp, zeros)
    us = u_v.reshape(_NC, _NPAD, _CH2)
    ds = dinv_v.reshape(_NC, _NPAD, _CH2)
    out = _final_kernel(
        sv[0], sv[1], us[0], us[1], ds[0], ds[1], hs[0], hs[1]
    )
    return out[:_N]
